# baseline (speedup 1.0000x reference)
import functools

import jax
import jax.numpy as jnp
from jax.experimental import pallas as pl
from jax.experimental.pallas import tpu as pltpu

_BH = 64
_BW = 128


def _score_body(x_ref, ae_ref, am_ref, ad_ref, p2_ref, p4_ref,
                e_ref, m_ref, d_ref):
    c, bh, w = x_ref.shape[1:]
    n = c * bh
    xf = x_ref[0].reshape(n, w)
    def row_roll8(a, k):
        a3 = a.reshape(a.shape[0] // 8, 8, a.shape[1])
        return pltpu.roll(a3, 8 - k, 1).reshape(a.shape)

    hi = jax.lax.Precision.HIGHEST
    f32 = jnp.float32
    for half, (lo, hi_l) in enumerate(((0, _BW), (_BW, 2 * _BW))):
        x = xf[:, lo:hi_l]
        xw = x + pltpu.roll(x, _BW - 1, 1)
        t2 = xw + row_roll8(xw, 1)
        t4w = t2 + pltpu.roll(t2, _BW - 2, 1)
        t4 = t4w + row_roll8(t4w, 2)
        e_ref[0, :, lo:hi_l] = jnp.dot(ae_ref[...], x,
                                       preferred_element_type=f32)
        sm = jnp.dot(am_ref[...], t2, preferred_element_type=f32)
        sd = jnp.dot(ad_ref[...], t4, preferred_element_type=f32)
        pm = jnp.dot(sm, p2_ref[half], precision=hi,
                     preferred_element_type=f32)
        pd = jnp.dot(sd, p4_ref[half], precision=hi,
                     preferred_element_type=f32)
        if half == 0:
            m_ref[0] = pm
            d_ref[0] = pd
        else:
            m_ref[0] += pm
            d_ref[0] += pd


def _monotone_key(x):
    b = jax.lax.bitcast_convert_type(x, jnp.int32)
    return jnp.where(b >= 0, b, b ^ jnp.int32(0x7FFFFFFF))


def _select_mask_body(e_ref, m_ref, d_ref, mf_ref, eo_ref, mo_ref, do_ref):
    e = e_ref[...]
    m = m_ref[...]
    d = d_ref[...]
    n = e.size + m.size + d.size
    ke = _monotone_key(e)
    km = _monotone_key(m)
    kd = _monotone_key(d)

    mf = mf_ref[0]
    r = jnp.int32(n - 1) - jnp.minimum(mf, jnp.int32(n - 1))

    def count_less(cand):
        return (jnp.sum((ke < cand).astype(jnp.int32))
                + jnp.sum((km < cand).astype(jnp.int32))
                + jnp.sum((kd < cand).astype(jnp.int32)))

    cnt0 = count_less(jnp.int32(0))
    prefix0 = jnp.where(cnt0 <= r, jnp.int32(0), jnp.int32(-2147483648))

    def body(i, prefix):
        bit = jnp.int32(1) << (jnp.int32(30) - i)
        cand = prefix | bit
        cnt = count_less(cand)
        return jnp.where(cnt <= r, cand, prefix)

    mkey = jax.lax.fori_loop(0, 31, body, prefix0)
    bits = jnp.where(mkey >= 0, mkey, mkey ^ jnp.int32(0x7FFFFFFF))
    thresh = jax.lax.bitcast_convert_type(bits, jnp.float32)
    thresh = jnp.where(mf >= n, jnp.float32(0.0), thresh)

    eo_ref[...] = jnp.where(e < thresh, jnp.float32(0.0), e)
    mo_ref[...] = jnp.where(m < thresh, jnp.float32(0.0), m)
    do_ref[...] = jnp.where(d < thresh, jnp.float32(0.0), d)


def kernel(encoded_features, w_early, w_middle, w_deep, max_features):
    B, C, H, W = encoded_features.shape
    f32 = jnp.float32

    def _amat(wvec, rows, stride, scale):
        k = jnp.arange(C * _BH)
        return jnp.where((k[None, :] % _BH) == stride * jnp.arange(rows)[:, None],
                         (wvec * scale)[k // _BH][None, :], f32(0.0))

    ae = _amat(w_early, _BH, 1, 1.0)
    am = _amat(w_middle, _BH // 2, 2, 0.25)
    ad = _amat(w_deep, _BH // 4, 4, 0.0625)

    iw = jnp.arange(_BW)
    iv = jnp.arange(2)[:, None, None]
    p2 = (iv * _BW + iw[None, :, None]
          == 2 * jnp.arange(W // 2)[None, None, :]).astype(f32)
    p4 = (iv * _BW + iw[None, :, None]
          == 4 * jnp.arange(W // 4)[None, None, :]).astype(f32)

    grid = (B, H // _BH)
    e, m, d = pl.pallas_call(
        _score_body,
        grid=grid,
        in_specs=[
            pl.BlockSpec((1, C, _BH, W), lambda b, h: (b, 0, h, 0)),
            pl.BlockSpec((_BH, C * _BH), lambda b, h: (0, 0)),
            pl.BlockSpec((_BH // 2, C * _BH), lambda b, h: (0, 0)),
            pl.BlockSpec((_BH // 4, C * _BH), lambda b, h: (0, 0)),
            pl.BlockSpec((2, _BW, W // 2), lambda b, h: (0, 0, 0)),
            pl.BlockSpec((2, _BW, W // 4), lambda b, h: (0, 0, 0)),
        ],
        out_specs=[
            pl.BlockSpec((1, _BH, W), lambda b, h: (b, h, 0)),
            pl.BlockSpec((1, _BH // 2, W // 2), lambda b, h: (b, h, 0)),
            pl.BlockSpec((1, _BH // 4, W // 4), lambda b, h: (b, h, 0)),
        ],
        out_shape=[
            jax.ShapeDtypeStruct((B, H, W), f32),
            jax.ShapeDtypeStruct((B, H // 2, W // 2), f32),
            jax.ShapeDtypeStruct((B, H // 4, W // 4), f32),
        ],
    )(encoded_features, ae, am, ad, p2, p4)

    mf = jnp.asarray(max_features, jnp.int32).reshape(1)
    eo, mo, do_ = pl.pallas_call(
        _select_mask_body,
        in_specs=[
            pl.BlockSpec(memory_space=pltpu.VMEM),
            pl.BlockSpec(memory_space=pltpu.VMEM),
            pl.BlockSpec(memory_space=pltpu.VMEM),
            pl.BlockSpec(memory_space=pltpu.SMEM),
        ],
        out_specs=[
            pl.BlockSpec(memory_space=pltpu.VMEM),
            pl.BlockSpec(memory_space=pltpu.VMEM),
            pl.BlockSpec(memory_space=pltpu.VMEM),
        ],
        out_shape=[
            jax.ShapeDtypeStruct(e.shape, f32),
            jax.ShapeDtypeStruct(m.shape, f32),
            jax.ShapeDtypeStruct(d.shape, f32),
        ],
    )(e, m, d, mf)
    return (eo, mo, do_)


# speedup vs baseline: 1.3025x; 1.3025x over previous
import functools

import jax
import jax.numpy as jnp
from jax.experimental import pallas as pl
from jax.experimental.pallas import tpu as pltpu

_BH = 32
_BW = 128


def _score_body(x_ref, ae_ref, am_ref, ad_ref, p2_ref, p4_ref,
                e_ref, m_ref, d_ref):
    c, bh, w = x_ref.shape[1:]
    n = c * bh
    xf = x_ref[0].reshape(n, w)
    def row_roll8(a, k):
        a3 = a.reshape(a.shape[0] // 8, 8, a.shape[1])
        return pltpu.roll(a3, 8 - k, 1).reshape(a.shape)

    hi = jax.lax.Precision.HIGHEST
    f32 = jnp.float32
    for half, (lo, hi_l) in enumerate(((0, _BW), (_BW, 2 * _BW))):
        x = xf[:, lo:hi_l]
        xw = x + pltpu.roll(x, _BW - 1, 1)
        t2 = xw + row_roll8(xw, 1)
        t4w = t2 + pltpu.roll(t2, _BW - 2, 1)
        t4 = t4w + row_roll8(t4w, 2)
        e_ref[0, :, lo:hi_l] = jnp.dot(ae_ref[...], x,
                                       preferred_element_type=f32)
        sm = jnp.dot(am_ref[...], t2, preferred_element_type=f32)
        sd = jnp.dot(ad_ref[...], t4, preferred_element_type=f32)
        pm = jnp.dot(sm, p2_ref[half], precision=hi,
                     preferred_element_type=f32)
        pd = jnp.dot(sd, p4_ref[half], precision=hi,
                     preferred_element_type=f32)
        if half == 0:
            m_ref[0] = pm
            d_ref[0] = pd
        else:
            m_ref[0] += pm
            d_ref[0] += pd


def _monotone_key(x):
    b = jax.lax.bitcast_convert_type(x, jnp.int32)
    return jnp.where(b >= 0, b, b ^ jnp.int32(0x7FFFFFFF))


def _select_mask_body(e_ref, m_ref, d_ref, mf_ref, eo_ref, mo_ref, do_ref):
    e = e_ref[...]
    m = m_ref[...]
    d = d_ref[...]
    n = e.size + m.size + d.size
    ke = _monotone_key(e)
    km = _monotone_key(m)
    kd = _monotone_key(d)

    mf = mf_ref[0]
    r = jnp.int32(n - 1) - jnp.minimum(mf, jnp.int32(n - 1))

    def count_less(cand):
        return (jnp.sum((ke < cand).astype(jnp.int32))
                + jnp.sum((km < cand).astype(jnp.int32))
                + jnp.sum((kd < cand).astype(jnp.int32)))

    cnt0 = count_less(jnp.int32(0))
    prefix0 = jnp.where(cnt0 <= r, jnp.int32(0), jnp.int32(-2147483648))

    def body(i, prefix):
        bit = jnp.int32(1) << (jnp.int32(30) - i)
        cand = prefix | bit
        cnt = count_less(cand)
        return jnp.where(cnt <= r, cand, prefix)

    mkey = jax.lax.fori_loop(0, 31, body, prefix0)
    bits = jnp.where(mkey >= 0, mkey, mkey ^ jnp.int32(0x7FFFFFFF))
    thresh = jax.lax.bitcast_convert_type(bits, jnp.float32)
    thresh = jnp.where(mf >= n, jnp.float32(0.0), thresh)

    eo_ref[...] = jnp.where(e < thresh, jnp.float32(0.0), e)
    mo_ref[...] = jnp.where(m < thresh, jnp.float32(0.0), m)
    do_ref[...] = jnp.where(d < thresh, jnp.float32(0.0), d)


def kernel(encoded_features, w_early, w_middle, w_deep, max_features):
    B, C, H, W = encoded_features.shape
    f32 = jnp.float32

    def _amat(wvec, rows, stride, scale):
        k = jnp.arange(C * _BH)
        return jnp.where((k[None, :] % _BH) == stride * jnp.arange(rows)[:, None],
                         (wvec * scale)[k // _BH][None, :], f32(0.0))

    ae = _amat(w_early, _BH, 1, 1.0)
    am = _amat(w_middle, _BH // 2, 2, 0.25)
    ad = _amat(w_deep, _BH // 4, 4, 0.0625)

    iw = jnp.arange(_BW)
    iv = jnp.arange(2)[:, None, None]
    p2 = (iv * _BW + iw[None, :, None]
          == 2 * jnp.arange(W // 2)[None, None, :]).astype(f32)
    p4 = (iv * _BW + iw[None, :, None]
          == 4 * jnp.arange(W // 4)[None, None, :]).astype(f32)

    grid = (B, H // _BH)
    e, m, d = pl.pallas_call(
        _score_body,
        grid=grid,
        in_specs=[
            pl.BlockSpec((1, C, _BH, W), lambda b, h: (b, 0, h, 0)),
            pl.BlockSpec((_BH, C * _BH), lambda b, h: (0, 0)),
            pl.BlockSpec((_BH // 2, C * _BH), lambda b, h: (0, 0)),
            pl.BlockSpec((_BH // 4, C * _BH), lambda b, h: (0, 0)),
            pl.BlockSpec((2, _BW, W // 2), lambda b, h: (0, 0, 0)),
            pl.BlockSpec((2, _BW, W // 4), lambda b, h: (0, 0, 0)),
        ],
        out_specs=[
            pl.BlockSpec((1, _BH, W), lambda b, h: (b, h, 0)),
            pl.BlockSpec((1, _BH // 2, W // 2), lambda b, h: (b, h, 0)),
            pl.BlockSpec((1, _BH // 4, W // 4), lambda b, h: (b, h, 0)),
        ],
        out_shape=[
            jax.ShapeDtypeStruct((B, H, W), f32),
            jax.ShapeDtypeStruct((B, H // 2, W // 2), f32),
            jax.ShapeDtypeStruct((B, H // 4, W // 4), f32),
        ],
    )(encoded_features, ae, am, ad, p2, p4)

    mf = jnp.asarray(max_features, jnp.int32).reshape(1)
    eo, mo, do_ = pl.pallas_call(
        _select_mask_body,
        in_specs=[
            pl.BlockSpec(memory_space=pltpu.VMEM),
            pl.BlockSpec(memory_space=pltpu.VMEM),
            pl.BlockSpec(memory_space=pltpu.VMEM),
            pl.BlockSpec(memory_space=pltpu.SMEM),
        ],
        out_specs=[
            pl.BlockSpec(memory_space=pltpu.VMEM),
            pl.BlockSpec(memory_space=pltpu.VMEM),
            pl.BlockSpec(memory_space=pltpu.VMEM),
        ],
        out_shape=[
            jax.ShapeDtypeStruct(e.shape, f32),
            jax.ShapeDtypeStruct(m.shape, f32),
            jax.ShapeDtypeStruct(d.shape, f32),
        ],
    )(e, m, d, mf)
    return (eo, mo, do_)
